# Initial kernel scaffold; baseline (speedup 1.0000x reference)
import jax
import jax.numpy as jnp
from jax.experimental import pallas as pl


def kernel(program_ops, emb_table, W_ih, W_hh, b_ih, b_hh):
    raise NotImplementedError("write your pallas kernel here")


# speedup vs baseline: 3.9881x; 3.9881x over previous
import functools

import jax
import jax.numpy as jnp
from jax import lax
from jax.experimental import pallas as pl
from jax.experimental.pallas import tpu as pltpu
from jax.experimental.pallas import tpu_sc as plsc

B = 4096
T = 50
E = 64
H = 128
CHUNK = 128


def _sc_gather(table, idx):
    n = idx.shape[0]
    info = plsc.get_sparse_core_info()
    nw = info.num_cores * info.num_subcores
    per_w = n // nw
    n_chunks = per_w // CHUNK
    mesh = plsc.VectorSubcoreMesh(core_axis_name="c", subcore_axis_name="s")

    @functools.partial(
        pl.kernel,
        mesh=mesh,
        out_type=jax.ShapeDtypeStruct((n, E), jnp.float32),
        scratch_types=[
            pltpu.VMEM((CHUNK,), jnp.int32),
            pltpu.VMEM((CHUNK, E), jnp.float32),
            pltpu.SemaphoreType.DMA,
        ],
        compiler_params=pltpu.CompilerParams(use_tc_tiling_on_sc=False),
    )
    def gather_kernel(table_hbm, idx_hbm, out_hbm, idx_v, rows_v, sem):
        wid = lax.axis_index("s") * info.num_cores + lax.axis_index("c")
        w_base = wid * per_w

        def body(i, _):
            base = w_base + i * CHUNK
            pltpu.sync_copy(idx_hbm.at[pl.ds(base, CHUNK)], idx_v)
            pltpu.async_copy(table_hbm.at[idx_v], rows_v, sem).wait()
            pltpu.sync_copy(rows_v, out_hbm.at[pl.ds(base, CHUNK)])
            return 0

        lax.fori_loop(0, n_chunks, body, 0)

    return gather_kernel(table, idx)


def _lstm_body(x_ref, wih_ref, whh_ref, b_ref, out_ref, h_ref, c_ref):
    t = pl.program_id(0)

    @pl.when(t == 0)
    def _():
        h_ref[...] = jnp.zeros_like(h_ref)
        c_ref[...] = jnp.zeros_like(c_ref)

    x = x_ref[0]
    h = h_ref[...]
    gates = (
        jnp.dot(x, wih_ref[...], preferred_element_type=jnp.float32)
        + jnp.dot(h, whh_ref[...], preferred_element_type=jnp.float32)
        + b_ref[...]
    )
    i = jax.nn.sigmoid(gates[:, 0:H])
    f = jax.nn.sigmoid(gates[:, H : 2 * H])
    g = jnp.tanh(gates[:, 2 * H : 3 * H])
    o = jax.nn.sigmoid(gates[:, 3 * H : 4 * H])
    c = f * c_ref[...] + i * g
    h_new = o * jnp.tanh(c)
    c_ref[...] = c
    h_ref[...] = h_new

    @pl.when(t == T - 1)
    def _():
        out_ref[...] = h_new


def _lstm_tc(x, wih_t, whh_t, bias):
    return pl.pallas_call(
        _lstm_body,
        grid=(T,),
        in_specs=[
            pl.BlockSpec((1, B, E), lambda t: (t, 0, 0)),
            pl.BlockSpec((E, 4 * H), lambda t: (0, 0)),
            pl.BlockSpec((H, 4 * H), lambda t: (0, 0)),
            pl.BlockSpec((1, 4 * H), lambda t: (0, 0)),
        ],
        out_specs=pl.BlockSpec((B, H), lambda t: (0, 0)),
        out_shape=jax.ShapeDtypeStruct((B, H), jnp.float32),
        scratch_shapes=[
            pltpu.VMEM((B, H), jnp.float32),
            pltpu.VMEM((B, H), jnp.float32),
        ],
        compiler_params=pltpu.CompilerParams(
            dimension_semantics=("arbitrary",),
        ),
    )(x, wih_t, whh_t, bias)


def kernel(program_ops, emb_table, W_ih, W_hh, b_ih, b_hh):
    idx = program_ops.T.reshape(-1).astype(jnp.int32)
    emb = _sc_gather(emb_table, idx)
    x = emb.reshape(T, B, E)
    wih_t = W_ih.T
    whh_t = W_hh.T
    bias = (b_ih + b_hh).reshape(1, 4 * H)
    return _lstm_tc(x, wih_t, whh_t, bias)


# speedup vs baseline: 3.9891x; 1.0002x over previous
import functools

import jax
import jax.numpy as jnp
from jax import lax
from jax.experimental import pallas as pl
from jax.experimental.pallas import tpu as pltpu
from jax.experimental.pallas import tpu_sc as plsc

B = 4096
T = 50
E = 64
H = 128
CHUNK = 128


def _sc_gather(table, idx):
    n = idx.shape[0]
    info = plsc.get_sparse_core_info()
    nw = info.num_cores * info.num_subcores
    per_w = n // nw
    n_chunks = per_w // CHUNK
    mesh = plsc.VectorSubcoreMesh(core_axis_name="c", subcore_axis_name="s")

    @functools.partial(
        pl.kernel,
        mesh=mesh,
        out_type=jax.ShapeDtypeStruct((n, E), jnp.float32),
        scratch_types=[
            pltpu.VMEM((CHUNK,), jnp.int32),
            pltpu.VMEM((CHUNK, E), jnp.float32),
            pltpu.SemaphoreType.DMA,
        ],
        compiler_params=pltpu.CompilerParams(use_tc_tiling_on_sc=False),
    )
    def gather_kernel(table_hbm, idx_hbm, out_hbm, idx_v, rows_v, sem):
        wid = lax.axis_index("s") * info.num_cores + lax.axis_index("c")
        w_base = wid * per_w

        def body(i, _):
            base = w_base + i * CHUNK
            pltpu.sync_copy(idx_hbm.at[pl.ds(base, CHUNK)], idx_v)
            pltpu.async_copy(table_hbm.at[idx_v], rows_v, sem).wait()
            pltpu.sync_copy(rows_v, out_hbm.at[pl.ds(base, CHUNK)])
            return 0

        lax.fori_loop(0, n_chunks, body, 0)

    return gather_kernel(table, idx)


def _lstm_body(x_ref, wih_ref, whh_ref, b_ref, out_ref, h_ref, c_ref):
    t = pl.program_id(0)

    @pl.when(t == 0)
    def _():
        h_ref[...] = jnp.zeros_like(h_ref)
        c_ref[...] = jnp.zeros_like(c_ref)

    x = x_ref[0].astype(jnp.bfloat16)
    h = h_ref[...].astype(jnp.bfloat16)
    gates = (
        jnp.dot(x, wih_ref[...], preferred_element_type=jnp.float32)
        + jnp.dot(h, whh_ref[...], preferred_element_type=jnp.float32)
        + b_ref[...]
    )
    i = jax.nn.sigmoid(gates[:, 0:H])
    f = jax.nn.sigmoid(gates[:, H : 2 * H])
    g = jnp.tanh(gates[:, 2 * H : 3 * H])
    o = jax.nn.sigmoid(gates[:, 3 * H : 4 * H])
    c = f * c_ref[...] + i * g
    h_new = o * jnp.tanh(c)
    c_ref[...] = c
    h_ref[...] = h_new

    @pl.when(t == T - 1)
    def _():
        out_ref[...] = h_new


def _lstm_tc(x, wih_t, whh_t, bias):
    return pl.pallas_call(
        _lstm_body,
        grid=(T,),
        in_specs=[
            pl.BlockSpec((1, B, E), lambda t: (t, 0, 0)),
            pl.BlockSpec((E, 4 * H), lambda t: (0, 0)),
            pl.BlockSpec((H, 4 * H), lambda t: (0, 0)),
            pl.BlockSpec((1, 4 * H), lambda t: (0, 0)),
        ],
        out_specs=pl.BlockSpec((B, H), lambda t: (0, 0)),
        out_shape=jax.ShapeDtypeStruct((B, H), jnp.float32),
        scratch_shapes=[
            pltpu.VMEM((B, H), jnp.float32),
            pltpu.VMEM((B, H), jnp.float32),
        ],
        compiler_params=pltpu.CompilerParams(
            dimension_semantics=("arbitrary",),
        ),
    )(x, wih_t, whh_t, bias)


def kernel(program_ops, emb_table, W_ih, W_hh, b_ih, b_hh):
    idx = program_ops.T.reshape(-1).astype(jnp.int32)
    emb = _sc_gather(emb_table, idx)
    x = emb.reshape(T, B, E)
    wih_t = W_ih.T.astype(jnp.bfloat16)
    whh_t = W_hh.T.astype(jnp.bfloat16)
    bias = (b_ih + b_hh).reshape(1, 4 * H)
    return _lstm_tc(x, wih_t, whh_t, bias)


# speedup vs baseline: 4.1662x; 1.0444x over previous
import functools

import jax
import jax.numpy as jnp
from jax import lax
from jax.experimental import pallas as pl
from jax.experimental.pallas import tpu as pltpu
from jax.experimental.pallas import tpu_sc as plsc

B = 4096
T = 50
E = 64
H = 128
CHUNK = 128


def _sc_gather(table, idx):
    n = idx.shape[0]
    info = plsc.get_sparse_core_info()
    nw = info.num_cores * info.num_subcores
    per_w = n // nw
    n_chunks = per_w // CHUNK
    mesh = plsc.VectorSubcoreMesh(core_axis_name="c", subcore_axis_name="s")

    @functools.partial(
        pl.kernel,
        mesh=mesh,
        out_type=jax.ShapeDtypeStruct((n, E), jnp.float32),
        scratch_types=[
            pltpu.VMEM((CHUNK,), jnp.int32),
            pltpu.VMEM((CHUNK, E), jnp.float32),
            pltpu.SemaphoreType.DMA,
        ],
        compiler_params=pltpu.CompilerParams(use_tc_tiling_on_sc=False),
    )
    def gather_kernel(table_hbm, idx_hbm, out_hbm, idx_v, rows_v, sem):
        wid = lax.axis_index("s") * info.num_cores + lax.axis_index("c")
        w_base = wid * per_w

        def body(i, _):
            base = w_base + i * CHUNK
            pltpu.sync_copy(idx_hbm.at[pl.ds(base, CHUNK)], idx_v)
            pltpu.async_copy(table_hbm.at[idx_v], rows_v, sem).wait()
            pltpu.sync_copy(rows_v, out_hbm.at[pl.ds(base, CHUNK)])
            return 0

        lax.fori_loop(0, n_chunks, body, 0)

    return gather_kernel(table, idx)


def _lstm_body(x_ref, wih_ref, whh_ref, b_ref, out_ref, h_ref, c_ref):
    t = pl.program_id(0)

    @pl.when(t == 0)
    def _():
        h_ref[...] = jnp.zeros_like(h_ref)
        c_ref[...] = jnp.zeros_like(c_ref)

    x = x_ref[0].astype(jnp.bfloat16)
    h = h_ref[...]
    gates = (
        jnp.dot(x, wih_ref[...], preferred_element_type=jnp.float32)
        + jnp.dot(h, whh_ref[...], preferred_element_type=jnp.float32)
        + b_ref[...]
    )

    def _sigmoid(z):
        return 0.5 + 0.5 * jnp.tanh(0.5 * z)

    i = _sigmoid(gates[:, 0:H])
    f = _sigmoid(gates[:, H : 2 * H])
    g = jnp.tanh(gates[:, 2 * H : 3 * H])
    o = _sigmoid(gates[:, 3 * H : 4 * H])
    c = f * c_ref[...] + i * g
    h_new = o * jnp.tanh(c)
    c_ref[...] = c
    h_ref[...] = h_new.astype(jnp.bfloat16)

    @pl.when(t == T - 1)
    def _():
        out_ref[...] = h_new


def _lstm_tc(x, wih_t, whh_t, bias):
    return pl.pallas_call(
        _lstm_body,
        grid=(T,),
        in_specs=[
            pl.BlockSpec((1, B, E), lambda t: (t, 0, 0)),
            pl.BlockSpec((E, 4 * H), lambda t: (0, 0)),
            pl.BlockSpec((H, 4 * H), lambda t: (0, 0)),
            pl.BlockSpec((1, 4 * H), lambda t: (0, 0)),
        ],
        out_specs=pl.BlockSpec((B, H), lambda t: (0, 0)),
        out_shape=jax.ShapeDtypeStruct((B, H), jnp.float32),
        scratch_shapes=[
            pltpu.VMEM((B, H), jnp.bfloat16),
            pltpu.VMEM((B, H), jnp.float32),
        ],
        compiler_params=pltpu.CompilerParams(
            dimension_semantics=("arbitrary",),
        ),
    )(x, wih_t, whh_t, bias)


def kernel(program_ops, emb_table, W_ih, W_hh, b_ih, b_hh):
    idx = program_ops.T.reshape(-1).astype(jnp.int32)
    emb = _sc_gather(emb_table, idx)
    x = emb.reshape(T, B, E)
    wih_t = W_ih.T.astype(jnp.bfloat16)
    whh_t = W_hh.T.astype(jnp.bfloat16)
    bias = (b_ih + b_hh).reshape(1, 4 * H)
    return _lstm_tc(x, wih_t, whh_t, bias)


# speedup vs baseline: 5.7404x; 1.3779x over previous
import functools

import jax
import jax.numpy as jnp
from jax import lax
from jax.experimental import pallas as pl
from jax.experimental.pallas import tpu as pltpu
from jax.experimental.pallas import tpu_sc as plsc

B = 4096
T = 50
E = 64
H = 128
CHUNK = 128


def _sc_gather(table, idx):
    n = idx.shape[0]
    info = plsc.get_sparse_core_info()
    nw = info.num_cores * info.num_subcores
    per_w = n // nw
    n_chunks = per_w // CHUNK
    n_pairs = n_chunks // 2
    mesh = plsc.VectorSubcoreMesh(core_axis_name="c", subcore_axis_name="s")

    @functools.partial(
        pl.kernel,
        mesh=mesh,
        out_type=jax.ShapeDtypeStruct((T, B, 128), jnp.float32),
        scratch_types=[
            pltpu.VMEM((per_w,), jnp.int32),
            pltpu.VMEM((CHUNK, E), jnp.float32),
            pltpu.VMEM((CHUNK, E), jnp.float32),
            pltpu.SemaphoreType.DMA,
            pltpu.SemaphoreType.DMA,
            pltpu.SemaphoreType.DMA,
            pltpu.SemaphoreType.DMA,
        ],
        compiler_params=pltpu.CompilerParams(use_tc_tiling_on_sc=False),
    )
    def gather_kernel(table_hbm, idx_hbm, out_hbm, idx_v, rows_a, rows_b,
                      sga, sgb, swa, swb):
        wid = lax.axis_index("s") * info.num_cores + lax.axis_index("c")
        base = wid * per_w
        pltpu.sync_copy(idx_hbm.at[pl.ds(base, per_w)], idx_v)

        def start_g(j, rows, sem):
            return pltpu.async_copy(
                table_hbm.at[idx_v.at[pl.ds(j * CHUNK, CHUNK)]], rows, sem)

        def out_slice(j):
            f = base + j * CHUNK
            t = f // B
            b = f % B
            return out_hbm.at[t, pl.ds(b, CHUNK), pl.ds(0, E)]

        def start_w(j, rows, sem):
            return pltpu.async_copy(rows, out_slice(j), sem)

        def wait_w(j, rows, sem):
            pltpu.make_async_copy(rows, out_slice(j), sem).wait()

        def body(i, _):
            j = 2 * i
            start_g(j, rows_a, sga).wait()

            @pl.when(i > 0)
            def _():
                wait_w(j - 1, rows_b, swb)

            start_w(j, rows_a, swa)
            start_g(j + 1, rows_b, sgb).wait()
            wait_w(j, rows_a, swa)
            start_w(j + 1, rows_b, swb)
            return 0

        lax.fori_loop(0, n_pairs, body, 0)
        wait_w(n_chunks - 1, rows_b, swb)

    return gather_kernel(table, idx)


def _lstm_body(x_ref, wih_ref, whh_ref, b_ref, out_ref, h_ref, c_ref):
    t = pl.program_id(0)

    @pl.when(t == 0)
    def _():
        h_ref[...] = jnp.zeros_like(h_ref)
        c_ref[...] = jnp.zeros_like(c_ref)

    x = x_ref[0, :, 0:E].astype(jnp.bfloat16)
    h = h_ref[...]
    gates = (
        jnp.dot(x, wih_ref[...], preferred_element_type=jnp.float32)
        + jnp.dot(h, whh_ref[...], preferred_element_type=jnp.float32)
        + b_ref[...]
    )

    def _sigmoid(z):
        return 0.5 + 0.5 * jnp.tanh(0.5 * z)

    i = _sigmoid(gates[:, 0:H])
    f = _sigmoid(gates[:, H : 2 * H])
    g = jnp.tanh(gates[:, 2 * H : 3 * H])
    o = _sigmoid(gates[:, 3 * H : 4 * H])
    c = f * c_ref[...] + i * g
    h_new = o * jnp.tanh(c)
    c_ref[...] = c
    h_ref[...] = h_new.astype(jnp.bfloat16)

    @pl.when(t == T - 1)
    def _():
        out_ref[...] = h_new


def _lstm_tc(x, wih_t, whh_t, bias):
    return pl.pallas_call(
        _lstm_body,
        grid=(T,),
        in_specs=[
            pl.BlockSpec((1, B, 128), lambda t: (t, 0, 0)),
            pl.BlockSpec((E, 4 * H), lambda t: (0, 0)),
            pl.BlockSpec((H, 4 * H), lambda t: (0, 0)),
            pl.BlockSpec((1, 4 * H), lambda t: (0, 0)),
        ],
        out_specs=pl.BlockSpec((B, H), lambda t: (0, 0)),
        out_shape=jax.ShapeDtypeStruct((B, H), jnp.float32),
        scratch_shapes=[
            pltpu.VMEM((B, H), jnp.bfloat16),
            pltpu.VMEM((B, H), jnp.float32),
        ],
        compiler_params=pltpu.CompilerParams(
            dimension_semantics=("arbitrary",),
        ),
    )(x, wih_t, whh_t, bias)


def kernel(program_ops, emb_table, W_ih, W_hh, b_ih, b_hh):
    idx = program_ops.T.reshape(-1).astype(jnp.int32)
    x = _sc_gather(emb_table, idx)
    wih_t = W_ih.T.astype(jnp.bfloat16)
    whh_t = W_hh.T.astype(jnp.bfloat16)
    bias = (b_ih + b_hh).reshape(1, 4 * H)
    return _lstm_tc(x, wih_t, whh_t, bias)


# speedup vs baseline: 6.1187x; 1.0659x over previous
import functools

import jax
import jax.numpy as jnp
from jax import lax
from jax.experimental import pallas as pl
from jax.experimental.pallas import tpu as pltpu
from jax.experimental.pallas import tpu_sc as plsc

B = 4096
T = 50
E = 64
H = 128
CHUNK = 128
NSEG = 5
TS = T // NSEG


def _sc_gather(table, idx, n_t):
    n = idx.shape[0]
    info = plsc.get_sparse_core_info()
    nw = info.num_cores * info.num_subcores
    per_w = n // nw
    n_chunks = per_w // CHUNK
    n_pairs = n_chunks // 2
    mesh = plsc.VectorSubcoreMesh(core_axis_name="c", subcore_axis_name="s")

    @functools.partial(
        pl.kernel,
        mesh=mesh,
        out_type=jax.ShapeDtypeStruct((n_t, B, 128), jnp.float32),
        scratch_types=[
            pltpu.VMEM((per_w,), jnp.int32),
            pltpu.VMEM((CHUNK, E), jnp.float32),
            pltpu.VMEM((CHUNK, E), jnp.float32),
            pltpu.SemaphoreType.DMA,
            pltpu.SemaphoreType.DMA,
            pltpu.SemaphoreType.DMA,
            pltpu.SemaphoreType.DMA,
        ],
        compiler_params=pltpu.CompilerParams(use_tc_tiling_on_sc=False),
    )
    def gather_kernel(table_hbm, idx_hbm, out_hbm, idx_v, rows_a, rows_b,
                      sga, sgb, swa, swb):
        wid = lax.axis_index("s") * info.num_cores + lax.axis_index("c")
        base = wid * per_w
        pltpu.sync_copy(idx_hbm.at[pl.ds(base, per_w)], idx_v)

        def start_g(j, rows, sem):
            return pltpu.async_copy(
                table_hbm.at[idx_v.at[pl.ds(j * CHUNK, CHUNK)]], rows, sem)

        def out_slice(j):
            f = base + j * CHUNK
            return out_hbm.at[f // B, pl.ds(f % B, CHUNK), pl.ds(0, E)]

        def start_w(j, rows, sem):
            return pltpu.async_copy(rows, out_slice(j), sem)

        def wait_w(j, rows, sem):
            pltpu.make_async_copy(rows, out_slice(j), sem).wait()

        def body(i, _):
            j = 2 * i
            start_g(j, rows_a, sga).wait()

            @pl.when(i > 0)
            def _():
                wait_w(j - 1, rows_b, swb)

            start_w(j, rows_a, swa)
            start_g(j + 1, rows_b, sgb).wait()
            wait_w(j, rows_a, swa)
            start_w(j + 1, rows_b, swb)
            return 0

        lax.fori_loop(0, n_pairs, body, 0)
        wait_w(n_chunks - 1, rows_b, swb)

    return gather_kernel(table, idx)


def _lstm_body(x_ref, wih_ref, whh_ref, b_ref, hin_ref, cin_ref,
               hout_ref, cout_ref, h_ref, c_ref, *, n_t):
    t = pl.program_id(0)

    @pl.when(t == 0)
    def _():
        h_ref[...] = hin_ref[...].astype(jnp.bfloat16)
        c_ref[...] = cin_ref[...]

    x = x_ref[0, :, 0:E].astype(jnp.bfloat16)
    h = h_ref[...]
    gates = (
        jnp.dot(x, wih_ref[...], preferred_element_type=jnp.float32)
        + jnp.dot(h, whh_ref[...], preferred_element_type=jnp.float32)
        + b_ref[...]
    )

    def _sigmoid(z):
        return 0.5 + 0.5 * jnp.tanh(0.5 * z)

    i = _sigmoid(gates[:, 0:H])
    f = _sigmoid(gates[:, H : 2 * H])
    g = jnp.tanh(gates[:, 2 * H : 3 * H])
    o = _sigmoid(gates[:, 3 * H : 4 * H])
    c = f * c_ref[...] + i * g
    h_new = o * jnp.tanh(c)
    c_ref[...] = c
    h_ref[...] = h_new.astype(jnp.bfloat16)

    @pl.when(t == n_t - 1)
    def _():
        hout_ref[...] = h_new
        cout_ref[...] = c


def _lstm_tc(x, wih_t, whh_t, bias, h_in, c_in):
    n_t = x.shape[0]
    return pl.pallas_call(
        functools.partial(_lstm_body, n_t=n_t),
        grid=(n_t,),
        in_specs=[
            pl.BlockSpec((1, B, 128), lambda t: (t, 0, 0)),
            pl.BlockSpec((E, 4 * H), lambda t: (0, 0)),
            pl.BlockSpec((H, 4 * H), lambda t: (0, 0)),
            pl.BlockSpec((1, 4 * H), lambda t: (0, 0)),
            pl.BlockSpec((B, H), lambda t: (0, 0)),
            pl.BlockSpec((B, H), lambda t: (0, 0)),
        ],
        out_specs=[
            pl.BlockSpec((B, H), lambda t: (0, 0)),
            pl.BlockSpec((B, H), lambda t: (0, 0)),
        ],
        out_shape=[
            jax.ShapeDtypeStruct((B, H), jnp.float32),
            jax.ShapeDtypeStruct((B, H), jnp.float32),
        ],
        scratch_shapes=[
            pltpu.VMEM((B, H), jnp.bfloat16),
            pltpu.VMEM((B, H), jnp.float32),
        ],
        compiler_params=pltpu.CompilerParams(
            dimension_semantics=("arbitrary",),
        ),
    )(x, wih_t, whh_t, bias, h_in, c_in)


def kernel(program_ops, emb_table, W_ih, W_hh, b_ih, b_hh):
    idx = program_ops.T.reshape(-1).astype(jnp.int32)
    wih_t = W_ih.T.astype(jnp.bfloat16)
    whh_t = W_hh.T.astype(jnp.bfloat16)
    bias = (b_ih + b_hh).reshape(1, 4 * H)

    xs = [
        _sc_gather(emb_table, idx[s * TS * B : (s + 1) * TS * B], TS)
        for s in range(NSEG)
    ]
    h = jnp.zeros((B, H), jnp.float32)
    c = jnp.zeros((B, H), jnp.float32)
    for s in range(NSEG):
        h, c = _lstm_tc(xs[s], wih_t, whh_t, bias, h, c)
    return h


# speedup vs baseline: 6.2125x; 1.0153x over previous
import functools

import jax
import jax.numpy as jnp
from jax import lax
from jax.experimental import pallas as pl
from jax.experimental.pallas import tpu as pltpu
from jax.experimental.pallas import tpu_sc as plsc

B = 4096
T = 50
E = 64
H = 128
CHUNK = 128
NSEG = 5
TS = T // NSEG


def _sc_gather(table, idx, n_t):
    n = idx.shape[0]
    info = plsc.get_sparse_core_info()
    nw = info.num_cores * info.num_subcores
    per_w = n // nw
    n_chunks = per_w // CHUNK
    n_pairs = n_chunks // 2
    mesh = plsc.VectorSubcoreMesh(core_axis_name="c", subcore_axis_name="s")

    @functools.partial(
        pl.kernel,
        mesh=mesh,
        out_type=jax.ShapeDtypeStruct((n_t, B, 128), jnp.float32),
        scratch_types=[
            pltpu.VMEM((per_w,), jnp.int32),
            pltpu.VMEM((CHUNK, 128), jnp.float32),
            pltpu.VMEM((CHUNK, 128), jnp.float32),
            pltpu.SemaphoreType.DMA,
            pltpu.SemaphoreType.DMA,
            pltpu.SemaphoreType.DMA,
            pltpu.SemaphoreType.DMA,
        ],
        compiler_params=pltpu.CompilerParams(use_tc_tiling_on_sc=False),
    )
    def gather_kernel(table_hbm, idx_hbm, out_hbm, idx_v, rows_a, rows_b,
                      sga, sgb, swa, swb):
        wid = lax.axis_index("s") * info.num_cores + lax.axis_index("c")
        base = wid * per_w
        pltpu.sync_copy(idx_hbm.at[pl.ds(base, per_w)], idx_v)

        def start_g(j, rows, sem):
            return pltpu.async_copy(
                table_hbm.at[idx_v.at[pl.ds(j * CHUNK, CHUNK)]], rows, sem)

        def out_slice(j):
            f = base + j * CHUNK
            return out_hbm.at[f // B, pl.ds(f % B, CHUNK), pl.ds(0, 128)]

        def start_w(j, rows, sem):
            return pltpu.async_copy(rows, out_slice(j), sem)

        def wait_w(j, rows, sem):
            pltpu.make_async_copy(rows, out_slice(j), sem).wait()

        def body(i, _):
            j = 2 * i
            start_g(j, rows_a, sga).wait()

            @pl.when(i > 0)
            def _():
                wait_w(j - 1, rows_b, swb)

            start_w(j, rows_a, swa)
            start_g(j + 1, rows_b, sgb).wait()
            wait_w(j, rows_a, swa)
            start_w(j + 1, rows_b, swb)
            return 0

        lax.fori_loop(0, n_pairs, body, 0)
        wait_w(n_chunks - 1, rows_b, swb)

    return gather_kernel(table, idx)


def _lstm_body(x_ref, w1_ref, w2_ref, hin_ref, cin_ref,
               hout_ref, cout_ref, h_ref, c_ref, *, n_t):
    t = pl.program_id(0)

    @pl.when(t == 0)
    def _():
        h_ref[...] = hin_ref[...].astype(jnp.bfloat16)
        c_ref[...] = cin_ref[...]

    x = x_ref[0].astype(jnp.bfloat16)
    h = h_ref[...]
    gates = (
        jnp.dot(x, w1_ref[...], preferred_element_type=jnp.float32)
        + jnp.dot(h, w2_ref[...], preferred_element_type=jnp.float32)
    )

    def _sigmoid(z):
        return 0.5 + 0.5 * jnp.tanh(z)

    i = _sigmoid(gates[:, 0:H])
    f = _sigmoid(gates[:, H : 2 * H])
    g = jnp.tanh(gates[:, 2 * H : 3 * H])
    o = _sigmoid(gates[:, 3 * H : 4 * H])
    c = f * c_ref[...] + i * g
    h_new = o * jnp.tanh(c)
    c_ref[...] = c
    h_ref[...] = h_new.astype(jnp.bfloat16)

    @pl.when(t == n_t - 1)
    def _():
        hout_ref[...] = h_new
        cout_ref[...] = c


def _lstm_tc(x, w1, w2, h_in, c_in):
    n_t = x.shape[0]
    return pl.pallas_call(
        functools.partial(_lstm_body, n_t=n_t),
        grid=(n_t,),
        in_specs=[
            pl.BlockSpec((1, B, 128), lambda t: (t, 0, 0)),
            pl.BlockSpec((128, 4 * H), lambda t: (0, 0)),
            pl.BlockSpec((H, 4 * H), lambda t: (0, 0)),
            pl.BlockSpec((B, H), lambda t: (0, 0)),
            pl.BlockSpec((B, H), lambda t: (0, 0)),
        ],
        out_specs=[
            pl.BlockSpec((B, H), lambda t: (0, 0)),
            pl.BlockSpec((B, H), lambda t: (0, 0)),
        ],
        out_shape=[
            jax.ShapeDtypeStruct((B, H), jnp.float32),
            jax.ShapeDtypeStruct((B, H), jnp.float32),
        ],
        scratch_shapes=[
            pltpu.VMEM((B, H), jnp.bfloat16),
            pltpu.VMEM((B, H), jnp.float32),
        ],
        compiler_params=pltpu.CompilerParams(
            dimension_semantics=("arbitrary",),
        ),
    )(x, w1, w2, h_in, c_in)


def kernel(program_ops, emb_table, W_ih, W_hh, b_ih, b_hh):
    n_ops = emb_table.shape[0]
    idx = program_ops.T.reshape(-1).astype(jnp.int32)

    table_p = jnp.concatenate(
        [
            emb_table,
            jnp.ones((n_ops, 2), jnp.float32),
            jnp.zeros((n_ops, 126 - E), jnp.float32),
        ],
        axis=1,
    )

    scale = jnp.concatenate(
        [jnp.full((H,), 0.5), jnp.full((H,), 0.5), jnp.ones((H,)),
         jnp.full((H,), 0.5)]
    ).astype(jnp.float32)
    bias = (b_ih + b_hh) * scale
    bias_hi = bias.astype(jnp.bfloat16).astype(jnp.float32)
    w1 = jnp.concatenate(
        [W_ih.T * scale[None, :], bias_hi[None, :], (bias - bias_hi)[None, :],
         jnp.zeros((126 - E, 4 * H), jnp.float32)],
        axis=0,
    ).astype(jnp.bfloat16)
    w2 = (W_hh.T * scale[None, :]).astype(jnp.bfloat16)

    xs = [
        _sc_gather(table_p, idx[s * TS * B : (s + 1) * TS * B], TS)
        for s in range(NSEG)
    ]
    h = jnp.zeros((B, H), jnp.float32)
    c = jnp.zeros((B, H), jnp.float32)
    for s in range(NSEG):
        h, c = _lstm_tc(xs[s], w1, w2, h, c)
    return h
